# Initial kernel scaffold; baseline (speedup 1.0000x reference)
import jax
import jax.numpy as jnp
from jax.experimental import pallas as pl


def kernel(x, edge_index, W_l, b_l, W_r):
    raise NotImplementedError("write your pallas kernel here")


# speedup vs baseline: 6.0856x; 6.0856x over previous
import functools

import jax
import jax.numpy as jnp
from jax import lax
from jax.experimental import pallas as pl
from jax.experimental.pallas import tpu as pltpu
from jax.experimental.pallas import tpu_sc as plsc

NC = 2
NS = 16
LANES = 16
CHUNK = 80


def _sc_aggregate(n_pad, e_per_w, d):
  rows_per_tile = n_pad // NS
  n_chunks = e_per_w // CHUNK
  zero_copies = rows_per_tile // CHUNK
  mesh = plsc.VectorSubcoreMesh(
      core_axis_name="c", subcore_axis_name="s", num_cores=NC, num_subcores=NS)

  @functools.partial(
      pl.kernel,
      out_type=[
          jax.ShapeDtypeStruct((NC, n_pad, d), jnp.float32),
          jax.ShapeDtypeStruct((NC, n_pad), jnp.float32),
      ],
      mesh=mesh,
      scratch_types=[
          pltpu.VMEM((CHUNK,), jnp.int32),
          pltpu.VMEM((CHUNK,), jnp.int32),
          pltpu.VMEM((CHUNK, d), jnp.float32),
          pltpu.VMEM((CHUNK,), jnp.float32),
          pltpu.VMEM((CHUNK,), jnp.float32),
          pltpu.VMEM_SHARED((n_pad, d), jnp.float32),
          pltpu.VMEM_SHARED((n_pad,), jnp.float32),
          pltpu.SemaphoreType.DMA,
      ],
  )
  def sc_agg(x_hbm, src_hbm, dst_hbm, part_out, cnt_out,
             sidx, didx, rows, ones, zcnt, acc_sh, cnt_sh, sem):
    cid = lax.axis_index("c")
    sid = lax.axis_index("s")
    wid = sid * NC + cid
    ebase = wid * e_per_w
    nbase = sid * rows_per_tile

    for j in range(CHUNK // LANES):
      sl = pl.ds(j * LANES, LANES)
      ones[sl] = jnp.full((LANES,), 1.0, jnp.float32)
      zcnt[sl] = jnp.zeros((LANES,), jnp.float32)

    @pl.loop(0, CHUNK)
    def _(i):
      for j in range(d // LANES):
        rows[i, pl.ds(j * LANES, LANES)] = jnp.zeros((LANES,), jnp.float32)

    @pl.loop(0, zero_copies)
    def _(i):
      off = nbase + i * CHUNK
      pltpu.sync_copy(rows, acc_sh.at[pl.ds(off, CHUNK)])
      pltpu.sync_copy(zcnt, cnt_sh.at[pl.ds(off, CHUNK)])

    plsc.subcore_barrier()

    @pl.loop(0, n_chunks)
    def _(k):
      off = ebase + k * CHUNK
      pltpu.sync_copy(src_hbm.at[pl.ds(off, CHUNK)], sidx)
      pltpu.sync_copy(dst_hbm.at[pl.ds(off, CHUNK)], didx)
      pltpu.async_copy(x_hbm.at[sidx], rows, sem).wait()
      pltpu.sync_copy(rows, acc_sh.at[didx], add=True)
      pltpu.sync_copy(ones, cnt_sh.at[didx], add=True)

    plsc.subcore_barrier()

    sl = pl.ds(nbase, rows_per_tile)
    pltpu.sync_copy(acc_sh.at[sl], part_out.at[cid, sl])
    pltpu.sync_copy(cnt_sh.at[sl], cnt_out.at[cid, sl])

  return sc_agg


def _tc_combine_body(p_ref, c_ref, x_ref, wl_ref, wr_ref, b_ref, o_ref):
  cnt = c_ref[0] + c_ref[1]
  inv = 1.0 / jnp.maximum(cnt, 1.0)
  agg = (p_ref[0] + p_ref[1]) * inv
  acc = jnp.dot(agg, wl_ref[...], preferred_element_type=jnp.float32)
  acc += jnp.dot(x_ref[...], wr_ref[...], preferred_element_type=jnp.float32)
  acc += b_ref[...]
  o_ref[...] = jnp.maximum(acc, 0.0)


def _tc_combine(part, cnt3, x, wl_t, wr_t, b2, n, d, br):
  grid = (n // br,)
  return pl.pallas_call(
      _tc_combine_body,
      grid=grid,
      in_specs=[
          pl.BlockSpec((NC, br, d), lambda i: (0, i, 0)),
          pl.BlockSpec((NC, br, 1), lambda i: (0, i, 0)),
          pl.BlockSpec((br, d), lambda i: (i, 0)),
          pl.BlockSpec((d, d), lambda i: (0, 0)),
          pl.BlockSpec((d, d), lambda i: (0, 0)),
          pl.BlockSpec((1, d), lambda i: (0, 0)),
      ],
      out_specs=pl.BlockSpec((br, d), lambda i: (i, 0)),
      out_shape=jax.ShapeDtypeStruct((n, d), jnp.float32),
  )(part, cnt3, x, wl_t, wr_t, b2)


def kernel(x, edge_index, W_l, b_l, W_r):
  n, d = x.shape
  e = edge_index.shape[1]
  rows_per_tile = -(-n // NS)
  rows_per_tile = -(-rows_per_tile // CHUNK) * CHUNK
  n_pad = rows_per_tile * NS
  e_per_w = -(-e // (NC * NS * CHUNK)) * CHUNK
  e_pad = e_per_w * NC * NS

  src = edge_index[0]
  dst = edge_index[1]
  if e_pad != e:
    pad = e_pad - e
    src = jnp.concatenate([src, jnp.zeros((pad,), jnp.int32)])
    dst = jnp.concatenate([dst, jnp.full((pad,), n_pad - 1, jnp.int32)])

  part, cnt = _sc_aggregate(n_pad, e_per_w, d)(x, src, dst)
  cnt3 = cnt.reshape(NC, n_pad, 1)
  br = 400
  return _tc_combine(part, cnt3, x, W_l.T, W_r.T, b_l[None, :], n, d, br)
